# baseline (speedup 1.0000x reference)
import functools

import jax
import jax.numpy as jnp
from jax import lax
from jax.experimental import pallas as pl
from jax.experimental.pallas import tpu as pltpu
from jax.experimental.pallas import tpu_sc as plsc

_K = 16
_QB = 128


_NEG = -3e38
_CH = 32


def _top16(vals, pos, m_sz):
    cols_v, cols_p = [], []
    cur = vals
    for _ in range(_K):
        m = jnp.max(cur, axis=0, keepdims=True)
        cp = jnp.min(jnp.where(cur >= m, pos, m_sz), axis=0, keepdims=True)
        cols_v.append(m)
        cols_p.append(cp)
        cur = jnp.where(pos == cp, _NEG, cur)
    return jnp.concatenate(cols_v, axis=0), jnp.concatenate(cols_p, axis=0)


def _knn2_body(keys_ref, q_ref, idx_ref, s3_ref, cm_ref, rv_ref, ri_ref):
    m_sz = keys_ref.shape[0]
    ql = q_ref.shape[1]
    cc = m_sz // _CH
    keys = keys_ref[...]
    q4 = q_ref[...]
    cross = jnp.dot(keys[:, :3], q4[:3, :],
                    preferred_element_type=jnp.float32)
    s2 = cross + keys[:, 3:4]
    s3_ref[...] = jnp.reshape(s2, (cc, _CH, ql))
    cm_ref[...] = jnp.max(s3_ref[...], axis=1)
    rv_ref[...] = jnp.full((_K, ql), _NEG, jnp.float32)
    ri_ref[...] = jnp.zeros((_K, ql), jnp.int32)
    sub_iota = lax.broadcasted_iota(jnp.int32, (cc, _CH, ql), 1)
    chunk_base = lax.broadcasted_iota(jnp.int32, (cc, ql), 0) * _CH

    def round_body(r, carry):
        cmr = cm_ref[...]
        rv = rv_ref[...]
        rmin = jnp.min(rv, axis=0, keepdims=True)
        amax = jnp.max(cmr - rmin)

        @pl.when(amax >= 0.0)
        def _go():
            s3 = s3_ref[...]
            hit = s3 >= cmr[:, None, :]
            posc = jnp.min(jnp.where(hit, sub_iota, _CH), axis=1)
            snew = jnp.where(sub_iota == posc[:, None, :], _NEG, s3)
            s3_ref[...] = snew
            cm_ref[...] = jnp.max(snew, axis=1)
            pos = chunk_base + posc
            hv, hi = _top16(jnp.concatenate([rv, cmr], axis=0),
                            jnp.concatenate([ri_ref[...], pos], axis=0), m_sz)
            rv_ref[...] = hv
            ri_ref[...] = hi

        return carry

    lax.fori_loop(0, _K, round_body, 0)
    idx_ref[...] = ri_ref[...]


def _knn2(keys4, q_aug):
    m = keys4.shape[0]
    n = q_aug.shape[1]
    cc = m // _CH
    return pl.pallas_call(
        _knn2_body,
        grid=(n // _QB,),
        in_specs=[
            pl.BlockSpec((m, 4), lambda i: (0, 0)),
            pl.BlockSpec((4, _QB), lambda i: (0, i)),
        ],
        out_specs=pl.BlockSpec((_K, _QB), lambda i: (0, i)),
        out_shape=jax.ShapeDtypeStruct((_K, n), jnp.int32),
        scratch_shapes=[
            pltpu.VMEM((cc, _CH, _QB), jnp.float32),
            pltpu.VMEM((cc, _QB), jnp.float32),
            pltpu.VMEM((_K, _QB), jnp.float32),
            pltpu.VMEM((_K, _QB), jnp.int32),
        ],
    )(keys4, q_aug)


def _knn_body(qt_ref, keys_ref, idx_ref):
    q = qt_ref[...]
    keys = keys_ref[...]
    m_sz = keys.shape[1]
    cross = jnp.dot(q, keys, preferred_element_type=jnp.float32)
    k2 = jnp.sum(keys * keys, axis=0, keepdims=True)
    q2 = jnp.sum(q * q, axis=1, keepdims=True)
    s = -((q2 + k2) - 2.0 * cross)
    iota = lax.broadcasted_iota(jnp.int32, (1, m_sz), 1)
    cols = []
    for _ in range(_K):
        m = jnp.max(s, axis=1, keepdims=True)
        hit = s >= m
        pos = jnp.min(jnp.where(hit, iota, m_sz), axis=1, keepdims=True)
        cols.append(pos)
        s = jnp.where(iota == pos, -3e38, s)
    idx_ref[...] = jnp.concatenate(cols, axis=1)


def _knn(query_t, keys):
    n = query_t.shape[0]
    m = keys.shape[1]
    return pl.pallas_call(
        _knn_body,
        grid=(n // _QB,),
        in_specs=[
            pl.BlockSpec((_QB, 3), lambda i: (i, 0)),
            pl.BlockSpec((3, m), lambda i: (0, 0)),
        ],
        out_specs=pl.BlockSpec((_QB, _K), lambda i: (i, 0)),
        out_shape=jax.ShapeDtypeStruct((n, _K), jnp.int32),
    )(query_t, keys)


def _sc_gather(table, idx):
    v, d = table.shape
    bn = idx.shape[0]
    info = plsc.get_sparse_core_info()
    nw = info.num_cores * info.num_subcores
    b_per_w = bn // nw
    ch = min(b_per_w, 512)
    n_ch = b_per_w // ch
    mesh = plsc.VectorSubcoreMesh(core_axis_name="c", subcore_axis_name="s")

    @functools.partial(
        pl.kernel, mesh=mesh,
        out_type=jax.ShapeDtypeStruct((bn, d), jnp.float32),
        compiler_params=pltpu.CompilerParams(use_tc_tiling_on_sc=False),
        scratch_types=[
            pltpu.VMEM((ch,), jnp.int32),
            pltpu.VMEM((ch, d), jnp.float32),
            pltpu.SemaphoreType.DMA,
        ],
    )
    def gk(table_hbm, idx_hbm, out_hbm, idx_v, rows_v, sem):
        wid = lax.axis_index("s") * info.num_cores + lax.axis_index("c")
        base = wid * b_per_w

        def body(i, carry):
            off = base + i * ch
            pltpu.sync_copy(idx_hbm.at[pl.ds(off, ch)], idx_v)
            pltpu.async_copy(table_hbm.at[idx_v], rows_v, sem).wait()
            pltpu.sync_copy(rows_v, out_hbm.at[pl.ds(off, ch)])
            return carry

        lax.fori_loop(0, n_ch, body, 0)

    return gk(table, idx)


def _leaky(x):
    return jnp.where(x >= 0, x, 0.01 * x)


def _relu(x):
    return jnp.maximum(x, 0.0)


def _rep_k(x, qb):
    c = x.shape[1]
    return jnp.reshape(
        jnp.broadcast_to(x[:, None, :], (qb, _K, c)), (qb * _K, c))


def _p2n_body(g_ref, f1_ref, x1_ref,
              w1a_ref, w1b_ref, w1c_ref, b1_ref, w2_ref, b2_ref,
              v1_ref, c1_ref, v2_ref, c2_ref, v3_ref, c3_ref, out_ref):
    qb = f1_ref.shape[0]
    g = g_ref[...]
    f2 = g[:, :64]
    xyz = g[:, 64:67]
    x1 = x1_ref[...]
    xyzn = xyz - _rep_k(x1, qb)
    a1 = jnp.dot(f1_ref[...], w1a_ref[...],
                 preferred_element_type=jnp.float32)
    l1 = _leaky(_rep_k(a1, qb)
                + jnp.dot(f2, w1b_ref[...], preferred_element_type=jnp.float32)
                + jnp.dot(xyzn, w1c_ref[...], preferred_element_type=jnp.float32)
                + b1_ref[...])
    p2p = _leaky(jnp.dot(l1, w2_ref[...], preferred_element_type=jnp.float32)
                 + b2_ref[...])
    h = _relu(jnp.dot(xyzn, v1_ref[...], preferred_element_type=jnp.float32)
              + c1_ref[...])
    h = _relu(jnp.dot(h, v2_ref[...], preferred_element_type=jnp.float32)
              + c2_ref[...])
    wn = _relu(jnp.dot(h, v3_ref[...], preferred_element_type=jnp.float32)
               + c3_ref[...])
    prod = p2p * wn
    p2n = jnp.sum(jnp.reshape(prod, (qb, _K, 64)), axis=1)
    out_ref[...] = jnp.concatenate(
        [p2n, x1, jnp.zeros((qb, 13), jnp.float32)], axis=1)


def _p2n_call(g2, f1t, x1t, wts):
    n = f1t.shape[0]
    grid = (n // _QB,)
    full = lambda a: pl.BlockSpec(a.shape, lambda i: tuple(0 for _ in a.shape))
    return pl.pallas_call(
        _p2n_body,
        grid=grid,
        in_specs=[
            pl.BlockSpec((_QB * _K, 80), lambda i: (i, 0)),
            pl.BlockSpec((_QB, 64), lambda i: (i, 0)),
            pl.BlockSpec((_QB, 3), lambda i: (i, 0)),
        ] + [full(w) for w in wts],
        out_specs=pl.BlockSpec((_QB, 80), lambda i: (i, 0)),
        out_shape=jax.ShapeDtypeStruct((n, 80), jnp.float32),
    )(g2, f1t, x1t, *wts)


def _n2n_body(g_ref, x1_ref, u1_ref, d1_ref, u2_ref, d2_ref, u3_ref, d3_ref,
              out_ref):
    qb = x1_ref.shape[0]
    g = g_ref[...]
    p2n = g[:, :64]
    xyz = g[:, 64:67]
    xyzn = xyz - _rep_k(x1_ref[...], qb)
    h = _relu(jnp.dot(xyzn, u1_ref[...], preferred_element_type=jnp.float32)
              + d1_ref[...])
    h = _relu(jnp.dot(h, u2_ref[...], preferred_element_type=jnp.float32)
              + d2_ref[...])
    wn = _relu(jnp.dot(h, u3_ref[...], preferred_element_type=jnp.float32)
               + d3_ref[...])
    prod = wn * p2n
    out_ref[...] = jnp.sum(jnp.reshape(prod, (qb, _K, 64)), axis=1)


def _n2n_call(g1, x1t, wts):
    n = x1t.shape[0]
    grid = (n // _QB,)
    full = lambda a: pl.BlockSpec(a.shape, lambda i: tuple(0 for _ in a.shape))
    return pl.pallas_call(
        _n2n_body,
        grid=grid,
        in_specs=[
            pl.BlockSpec((_QB * _K, 80), lambda i: (i, 0)),
            pl.BlockSpec((_QB, 3), lambda i: (i, 0)),
        ] + [full(w) for w in wts],
        out_specs=pl.BlockSpec((_QB, 64), lambda i: (i, 0)),
        out_shape=jax.ShapeDtypeStruct((n, 64), jnp.float32),
    )(g1, x1t, *wts)


def kernel(xyz1, feat1, xyz2, feat2, cost_w1, cost_b1, cost_w2, cost_b2,
           wn1_w1, wn1_b1, wn1_w2, wn1_b2, wn1_w3, wn1_b3,
           wn2_w1, wn2_b1, wn2_w2, wn2_b2, wn2_w3, wn2_b3):
    B, C, N = feat1.shape
    x1t = jnp.transpose(xyz1[0])
    x2t = jnp.transpose(xyz2[0])

    def keys4(xt):
        return jnp.concatenate(
            [2.0 * xt, -jnp.sum(xt * xt, axis=1, keepdims=True)], axis=1)

    n_col = xyz1.shape[2]
    q_aug = jnp.concatenate(
        [xyz1[0], jnp.ones((1, n_col), jnp.float32)], axis=0)
    idx12 = jnp.transpose(_knn2(keys4(x2t), q_aug))
    idx11 = jnp.transpose(_knn2(keys4(x1t), q_aug))

    f2t = jnp.transpose(feat2[0])
    table2 = jnp.concatenate(
        [f2t, x2t, jnp.zeros((N, 13), jnp.float32)], axis=1)
    g2 = _sc_gather(table2, jnp.reshape(idx12, (-1,)))

    f1t = jnp.transpose(feat1[0])
    wts_c = (
        jnp.transpose(cost_w1[:, :64]),
        jnp.transpose(cost_w1[:, 64:128]),
        jnp.transpose(cost_w1[:, 128:131]),
        cost_b1[None, :],
        jnp.transpose(cost_w2), cost_b2[None, :],
        jnp.transpose(wn2_w1), wn2_b1[None, :],
        jnp.transpose(wn2_w2), wn2_b2[None, :],
        jnp.transpose(wn2_w3), wn2_b3[None, :],
    )
    table1 = _p2n_call(g2, f1t, x1t, wts_c)

    g1 = _sc_gather(table1, jnp.reshape(idx11, (-1,)))
    wts_n = (
        jnp.transpose(wn1_w1), wn1_b1[None, :],
        jnp.transpose(wn1_w2), wn1_b2[None, :],
        jnp.transpose(wn1_w3), wn1_b3[None, :],
    )
    n2n = _n2n_call(g1, x1t, wts_n)
    return jnp.transpose(n2n)[None]


# speedup vs baseline: 1.1093x; 1.1093x over previous
import functools

import jax
import jax.numpy as jnp
from jax import lax
from jax.experimental import pallas as pl
from jax.experimental.pallas import tpu as pltpu
from jax.experimental.pallas import tpu_sc as plsc

_K = 16
_QB = 128


_NEG = -3e38
_CH = 128


def _top16(vals, pos, m_sz):
    cols_v, cols_p = [], []
    cur = vals
    for _ in range(_K):
        m = jnp.max(cur, axis=0, keepdims=True)
        cp = jnp.min(jnp.where(cur >= m, pos, m_sz), axis=0, keepdims=True)
        cols_v.append(m)
        cols_p.append(cp)
        cur = jnp.where(pos == cp, _NEG, cur)
    return jnp.concatenate(cols_v, axis=0), jnp.concatenate(cols_p, axis=0)


def _knn2_body(keys_ref, q_ref, idx_ref, s3_ref, cm_ref, rv_ref, ri_ref):
    m_sz = keys_ref.shape[0]
    ql = q_ref.shape[1]
    cc = m_sz // _CH
    keys = keys_ref[...]
    q4 = q_ref[...]
    cross = jnp.dot(keys[:, :3], q4[:3, :],
                    preferred_element_type=jnp.float32)
    s2 = cross + keys[:, 3:4]
    s3_ref[...] = jnp.reshape(s2, (cc, _CH, ql))
    cm_ref[...] = jnp.max(s3_ref[...], axis=1)
    rv_ref[...] = jnp.full((_K, ql), _NEG, jnp.float32)
    ri_ref[...] = jnp.zeros((_K, ql), jnp.int32)
    sub_iota = lax.broadcasted_iota(jnp.int32, (cc, _CH, ql), 1)
    chunk_base = lax.broadcasted_iota(jnp.int32, (cc, ql), 0) * _CH

    def round_body(r, carry):
        cmr = cm_ref[...]
        rv = rv_ref[...]
        rmin = jnp.min(rv, axis=0, keepdims=True)
        amax = jnp.max(cmr - rmin)

        @pl.when(amax >= 0.0)
        def _go():
            s3 = s3_ref[...]
            hit = s3 >= cmr[:, None, :]
            posc = jnp.min(jnp.where(hit, sub_iota, _CH), axis=1)
            snew = jnp.where(sub_iota == posc[:, None, :], _NEG, s3)
            s3_ref[...] = snew
            cm_ref[...] = jnp.max(snew, axis=1)
            pos = chunk_base + posc
            hv, hi = _top16(jnp.concatenate([rv, cmr], axis=0),
                            jnp.concatenate([ri_ref[...], pos], axis=0), m_sz)
            rv_ref[...] = hv
            ri_ref[...] = hi

        return carry

    lax.fori_loop(0, _K, round_body, 0)
    idx_ref[...] = ri_ref[...]


def _knn2(keys4, q_aug):
    m = keys4.shape[0]
    n = q_aug.shape[1]
    cc = m // _CH
    return pl.pallas_call(
        _knn2_body,
        grid=(n // _QB,),
        in_specs=[
            pl.BlockSpec((m, 4), lambda i: (0, 0)),
            pl.BlockSpec((4, _QB), lambda i: (0, i)),
        ],
        out_specs=pl.BlockSpec((_K, _QB), lambda i: (0, i)),
        out_shape=jax.ShapeDtypeStruct((_K, n), jnp.int32),
        scratch_shapes=[
            pltpu.VMEM((cc, _CH, _QB), jnp.float32),
            pltpu.VMEM((cc, _QB), jnp.float32),
            pltpu.VMEM((_K, _QB), jnp.float32),
            pltpu.VMEM((_K, _QB), jnp.int32),
        ],
    )(keys4, q_aug)


def _knn_body(qt_ref, keys_ref, idx_ref):
    q = qt_ref[...]
    keys = keys_ref[...]
    m_sz = keys.shape[1]
    cross = jnp.dot(q, keys, preferred_element_type=jnp.float32)
    k2 = jnp.sum(keys * keys, axis=0, keepdims=True)
    q2 = jnp.sum(q * q, axis=1, keepdims=True)
    s = -((q2 + k2) - 2.0 * cross)
    iota = lax.broadcasted_iota(jnp.int32, (1, m_sz), 1)
    cols = []
    for _ in range(_K):
        m = jnp.max(s, axis=1, keepdims=True)
        hit = s >= m
        pos = jnp.min(jnp.where(hit, iota, m_sz), axis=1, keepdims=True)
        cols.append(pos)
        s = jnp.where(iota == pos, -3e38, s)
    idx_ref[...] = jnp.concatenate(cols, axis=1)


def _knn(query_t, keys):
    n = query_t.shape[0]
    m = keys.shape[1]
    return pl.pallas_call(
        _knn_body,
        grid=(n // _QB,),
        in_specs=[
            pl.BlockSpec((_QB, 3), lambda i: (i, 0)),
            pl.BlockSpec((3, m), lambda i: (0, 0)),
        ],
        out_specs=pl.BlockSpec((_QB, _K), lambda i: (i, 0)),
        out_shape=jax.ShapeDtypeStruct((n, _K), jnp.int32),
    )(query_t, keys)


def _sc_gather(table, idx):
    v, d = table.shape
    bn = idx.shape[0]
    info = plsc.get_sparse_core_info()
    nw = info.num_cores * info.num_subcores
    b_per_w = bn // nw
    ch = min(b_per_w, 512)
    n_ch = b_per_w // ch
    mesh = plsc.VectorSubcoreMesh(core_axis_name="c", subcore_axis_name="s")

    @functools.partial(
        pl.kernel, mesh=mesh,
        out_type=jax.ShapeDtypeStruct((bn, d), jnp.float32),
        compiler_params=pltpu.CompilerParams(use_tc_tiling_on_sc=False),
        scratch_types=[
            pltpu.VMEM((ch,), jnp.int32),
            pltpu.VMEM((ch, d), jnp.float32),
            pltpu.SemaphoreType.DMA,
        ],
    )
    def gk(table_hbm, idx_hbm, out_hbm, idx_v, rows_v, sem):
        wid = lax.axis_index("s") * info.num_cores + lax.axis_index("c")
        base = wid * b_per_w

        def body(i, carry):
            off = base + i * ch
            pltpu.sync_copy(idx_hbm.at[pl.ds(off, ch)], idx_v)
            pltpu.async_copy(table_hbm.at[idx_v], rows_v, sem).wait()
            pltpu.sync_copy(rows_v, out_hbm.at[pl.ds(off, ch)])
            return carry

        lax.fori_loop(0, n_ch, body, 0)

    return gk(table, idx)


def _leaky(x):
    return jnp.where(x >= 0, x, 0.01 * x)


def _relu(x):
    return jnp.maximum(x, 0.0)


def _rep_k(x, qb):
    c = x.shape[1]
    return jnp.reshape(
        jnp.broadcast_to(x[:, None, :], (qb, _K, c)), (qb * _K, c))


def _p2n_body(g_ref, f1_ref, x1_ref,
              w1a_ref, w1b_ref, w1c_ref, b1_ref, w2_ref, b2_ref,
              v1_ref, c1_ref, v2_ref, c2_ref, v3_ref, c3_ref, out_ref):
    qb = f1_ref.shape[0]
    g = g_ref[...]
    f2 = g[:, :64]
    xyz = g[:, 64:67]
    x1 = x1_ref[...]
    xyzn = xyz - _rep_k(x1, qb)
    a1 = jnp.dot(f1_ref[...], w1a_ref[...],
                 preferred_element_type=jnp.float32)
    l1 = _leaky(_rep_k(a1, qb)
                + jnp.dot(f2, w1b_ref[...], preferred_element_type=jnp.float32)
                + jnp.dot(xyzn, w1c_ref[...], preferred_element_type=jnp.float32)
                + b1_ref[...])
    p2p = _leaky(jnp.dot(l1, w2_ref[...], preferred_element_type=jnp.float32)
                 + b2_ref[...])
    h = _relu(jnp.dot(xyzn, v1_ref[...], preferred_element_type=jnp.float32)
              + c1_ref[...])
    h = _relu(jnp.dot(h, v2_ref[...], preferred_element_type=jnp.float32)
              + c2_ref[...])
    wn = _relu(jnp.dot(h, v3_ref[...], preferred_element_type=jnp.float32)
               + c3_ref[...])
    prod = p2p * wn
    p2n = jnp.sum(jnp.reshape(prod, (qb, _K, 64)), axis=1)
    out_ref[...] = jnp.concatenate(
        [p2n, x1, jnp.zeros((qb, 13), jnp.float32)], axis=1)


def _p2n_call(g2, f1t, x1t, wts):
    n = f1t.shape[0]
    grid = (n // _QB,)
    full = lambda a: pl.BlockSpec(a.shape, lambda i: tuple(0 for _ in a.shape))
    return pl.pallas_call(
        _p2n_body,
        grid=grid,
        in_specs=[
            pl.BlockSpec((_QB * _K, 80), lambda i: (i, 0)),
            pl.BlockSpec((_QB, 64), lambda i: (i, 0)),
            pl.BlockSpec((_QB, 3), lambda i: (i, 0)),
        ] + [full(w) for w in wts],
        out_specs=pl.BlockSpec((_QB, 80), lambda i: (i, 0)),
        out_shape=jax.ShapeDtypeStruct((n, 80), jnp.float32),
    )(g2, f1t, x1t, *wts)


def _n2n_body(g_ref, x1_ref, u1_ref, d1_ref, u2_ref, d2_ref, u3_ref, d3_ref,
              out_ref):
    qb = x1_ref.shape[0]
    g = g_ref[...]
    p2n = g[:, :64]
    xyz = g[:, 64:67]
    xyzn = xyz - _rep_k(x1_ref[...], qb)
    h = _relu(jnp.dot(xyzn, u1_ref[...], preferred_element_type=jnp.float32)
              + d1_ref[...])
    h = _relu(jnp.dot(h, u2_ref[...], preferred_element_type=jnp.float32)
              + d2_ref[...])
    wn = _relu(jnp.dot(h, u3_ref[...], preferred_element_type=jnp.float32)
               + d3_ref[...])
    prod = wn * p2n
    out_ref[...] = jnp.sum(jnp.reshape(prod, (qb, _K, 64)), axis=1)


def _n2n_call(g1, x1t, wts):
    n = x1t.shape[0]
    grid = (n // _QB,)
    full = lambda a: pl.BlockSpec(a.shape, lambda i: tuple(0 for _ in a.shape))
    return pl.pallas_call(
        _n2n_body,
        grid=grid,
        in_specs=[
            pl.BlockSpec((_QB * _K, 80), lambda i: (i, 0)),
            pl.BlockSpec((_QB, 3), lambda i: (i, 0)),
        ] + [full(w) for w in wts],
        out_specs=pl.BlockSpec((_QB, 64), lambda i: (i, 0)),
        out_shape=jax.ShapeDtypeStruct((n, 64), jnp.float32),
    )(g1, x1t, *wts)


def kernel(xyz1, feat1, xyz2, feat2, cost_w1, cost_b1, cost_w2, cost_b2,
           wn1_w1, wn1_b1, wn1_w2, wn1_b2, wn1_w3, wn1_b3,
           wn2_w1, wn2_b1, wn2_w2, wn2_b2, wn2_w3, wn2_b3):
    B, C, N = feat1.shape
    x1t = jnp.transpose(xyz1[0])
    x2t = jnp.transpose(xyz2[0])

    def keys4(xt):
        return jnp.concatenate(
            [2.0 * xt, -jnp.sum(xt * xt, axis=1, keepdims=True)], axis=1)

    n_col = xyz1.shape[2]
    q_aug = jnp.concatenate(
        [xyz1[0], jnp.ones((1, n_col), jnp.float32)], axis=0)
    idx12 = jnp.transpose(_knn2(keys4(x2t), q_aug))
    idx11 = jnp.transpose(_knn2(keys4(x1t), q_aug))

    f2t = jnp.transpose(feat2[0])
    table2 = jnp.concatenate(
        [f2t, x2t, jnp.zeros((N, 13), jnp.float32)], axis=1)
    g2 = _sc_gather(table2, jnp.reshape(idx12, (-1,)))

    f1t = jnp.transpose(feat1[0])
    wts_c = (
        jnp.transpose(cost_w1[:, :64]),
        jnp.transpose(cost_w1[:, 64:128]),
        jnp.transpose(cost_w1[:, 128:131]),
        cost_b1[None, :],
        jnp.transpose(cost_w2), cost_b2[None, :],
        jnp.transpose(wn2_w1), wn2_b1[None, :],
        jnp.transpose(wn2_w2), wn2_b2[None, :],
        jnp.transpose(wn2_w3), wn2_b3[None, :],
    )
    table1 = _p2n_call(g2, f1t, x1t, wts_c)

    g1 = _sc_gather(table1, jnp.reshape(idx11, (-1,)))
    wts_n = (
        jnp.transpose(wn1_w1), wn1_b1[None, :],
        jnp.transpose(wn1_w2), wn1_b2[None, :],
        jnp.transpose(wn1_w3), wn1_b3[None, :],
    )
    n2n = _n2n_call(g1, x1t, wts_n)
    return jnp.transpose(n2n)[None]


# speedup vs baseline: 1.3169x; 1.1872x over previous
import functools

import jax
import jax.numpy as jnp
from jax import lax
from jax.experimental import pallas as pl
from jax.experimental.pallas import tpu as pltpu
from jax.experimental.pallas import tpu_sc as plsc

_K = 16
_QB = 256


_NEG = -3e38
_CH = 128


def _top16(vals, pos, m_sz):
    cols_v, cols_p = [], []
    cur = vals
    for _ in range(_K):
        m = jnp.max(cur, axis=0, keepdims=True)
        cp = jnp.min(jnp.where(cur >= m, pos, m_sz), axis=0, keepdims=True)
        cols_v.append(m)
        cols_p.append(cp)
        cur = jnp.where(pos == cp, _NEG, cur)
    return jnp.concatenate(cols_v, axis=0), jnp.concatenate(cols_p, axis=0)


def _knn2_body(keys_ref, q_ref, idx_ref, s3_ref, cm_ref, rv_ref, ri_ref):
    m_sz = keys_ref.shape[0]
    ql = q_ref.shape[1]
    cc = m_sz // _CH
    keys = keys_ref[...]
    q4 = q_ref[...]
    cross = jnp.dot(keys[:, :3], q4[:3, :],
                    preferred_element_type=jnp.float32)
    s2 = cross + keys[:, 3:4]
    s3_ref[...] = jnp.reshape(s2, (cc, _CH, ql))
    cm_ref[...] = jnp.max(s3_ref[...], axis=1)
    rv_ref[...] = jnp.full((_K, ql), _NEG, jnp.float32)
    ri_ref[...] = jnp.zeros((_K, ql), jnp.int32)
    sub_iota = lax.broadcasted_iota(jnp.int32, (cc, _CH, ql), 1)
    chunk_base = lax.broadcasted_iota(jnp.int32, (cc, ql), 0) * _CH

    def round_body(r, carry):
        cmr = cm_ref[...]
        rv = rv_ref[...]
        rmin = jnp.min(rv, axis=0, keepdims=True)
        amax = jnp.max(cmr - rmin)

        @pl.when(amax >= 0.0)
        def _go():
            s3 = s3_ref[...]
            hit = s3 >= cmr[:, None, :]
            posc = jnp.min(jnp.where(hit, sub_iota, _CH), axis=1)
            snew = jnp.where(sub_iota == posc[:, None, :], _NEG, s3)
            s3_ref[...] = snew
            cm_ref[...] = jnp.max(snew, axis=1)
            pos = chunk_base + posc
            hv, hi = _top16(jnp.concatenate([rv, cmr], axis=0),
                            jnp.concatenate([ri_ref[...], pos], axis=0), m_sz)
            rv_ref[...] = hv
            ri_ref[...] = hi

        return carry

    lax.fori_loop(0, _K, round_body, 0)
    idx_ref[...] = ri_ref[...]


def _knn2(keys4, q_aug):
    m = keys4.shape[0]
    n = q_aug.shape[1]
    cc = m // _CH
    return pl.pallas_call(
        _knn2_body,
        grid=(n // _QB,),
        in_specs=[
            pl.BlockSpec((m, 4), lambda i: (0, 0)),
            pl.BlockSpec((4, _QB), lambda i: (0, i)),
        ],
        out_specs=pl.BlockSpec((_K, _QB), lambda i: (0, i)),
        out_shape=jax.ShapeDtypeStruct((_K, n), jnp.int32),
        scratch_shapes=[
            pltpu.VMEM((cc, _CH, _QB), jnp.float32),
            pltpu.VMEM((cc, _QB), jnp.float32),
            pltpu.VMEM((_K, _QB), jnp.float32),
            pltpu.VMEM((_K, _QB), jnp.int32),
        ],
    )(keys4, q_aug)


def _knn_body(qt_ref, keys_ref, idx_ref):
    q = qt_ref[...]
    keys = keys_ref[...]
    m_sz = keys.shape[1]
    cross = jnp.dot(q, keys, preferred_element_type=jnp.float32)
    k2 = jnp.sum(keys * keys, axis=0, keepdims=True)
    q2 = jnp.sum(q * q, axis=1, keepdims=True)
    s = -((q2 + k2) - 2.0 * cross)
    iota = lax.broadcasted_iota(jnp.int32, (1, m_sz), 1)
    cols = []
    for _ in range(_K):
        m = jnp.max(s, axis=1, keepdims=True)
        hit = s >= m
        pos = jnp.min(jnp.where(hit, iota, m_sz), axis=1, keepdims=True)
        cols.append(pos)
        s = jnp.where(iota == pos, -3e38, s)
    idx_ref[...] = jnp.concatenate(cols, axis=1)


def _knn(query_t, keys):
    n = query_t.shape[0]
    m = keys.shape[1]
    return pl.pallas_call(
        _knn_body,
        grid=(n // _QB,),
        in_specs=[
            pl.BlockSpec((_QB, 3), lambda i: (i, 0)),
            pl.BlockSpec((3, m), lambda i: (0, 0)),
        ],
        out_specs=pl.BlockSpec((_QB, _K), lambda i: (i, 0)),
        out_shape=jax.ShapeDtypeStruct((n, _K), jnp.int32),
    )(query_t, keys)


def _sc_gather(table, idx):
    v, d = table.shape
    bn = idx.shape[0]
    info = plsc.get_sparse_core_info()
    nw = info.num_cores * info.num_subcores
    b_per_w = bn // nw
    ch = min(b_per_w, 512)
    n_ch = b_per_w // ch
    mesh = plsc.VectorSubcoreMesh(core_axis_name="c", subcore_axis_name="s")

    @functools.partial(
        pl.kernel, mesh=mesh,
        out_type=jax.ShapeDtypeStruct((bn, d), jnp.float32),
        compiler_params=pltpu.CompilerParams(use_tc_tiling_on_sc=False),
        scratch_types=[
            pltpu.VMEM((ch,), jnp.int32),
            pltpu.VMEM((ch, d), jnp.float32),
            pltpu.SemaphoreType.DMA,
        ],
    )
    def gk(table_hbm, idx_hbm, out_hbm, idx_v, rows_v, sem):
        wid = lax.axis_index("s") * info.num_cores + lax.axis_index("c")
        base = wid * b_per_w

        def body(i, carry):
            off = base + i * ch
            pltpu.sync_copy(idx_hbm.at[pl.ds(off, ch)], idx_v)
            pltpu.async_copy(table_hbm.at[idx_v], rows_v, sem).wait()
            pltpu.sync_copy(rows_v, out_hbm.at[pl.ds(off, ch)])
            return carry

        lax.fori_loop(0, n_ch, body, 0)

    return gk(table, idx)


def _leaky(x):
    return jnp.where(x >= 0, x, 0.01 * x)


def _relu(x):
    return jnp.maximum(x, 0.0)


def _rep_k(x, qb):
    c = x.shape[1]
    return jnp.reshape(
        jnp.broadcast_to(x[:, None, :], (qb, _K, c)), (qb * _K, c))


def _p2n_body(g_ref, f1_ref, x1_ref,
              w1a_ref, w1b_ref, w1c_ref, b1_ref, w2_ref, b2_ref,
              v1_ref, c1_ref, v2_ref, c2_ref, v3_ref, c3_ref, out_ref):
    qb = f1_ref.shape[0]
    g = g_ref[...]
    f2 = g[:, :64]
    xyz = g[:, 64:67]
    x1 = x1_ref[...]
    xyzn = xyz - _rep_k(x1, qb)
    a1 = jnp.dot(f1_ref[...], w1a_ref[...],
                 preferred_element_type=jnp.float32)
    l1 = _leaky(_rep_k(a1, qb)
                + jnp.dot(f2, w1b_ref[...], preferred_element_type=jnp.float32)
                + jnp.dot(xyzn, w1c_ref[...], preferred_element_type=jnp.float32)
                + b1_ref[...])
    p2p = _leaky(jnp.dot(l1, w2_ref[...], preferred_element_type=jnp.float32)
                 + b2_ref[...])
    h = _relu(jnp.dot(xyzn, v1_ref[...], preferred_element_type=jnp.float32)
              + c1_ref[...])
    h = _relu(jnp.dot(h, v2_ref[...], preferred_element_type=jnp.float32)
              + c2_ref[...])
    wn = _relu(jnp.dot(h, v3_ref[...], preferred_element_type=jnp.float32)
               + c3_ref[...])
    prod = p2p * wn
    p2n = jnp.sum(jnp.reshape(prod, (qb, _K, 64)), axis=1)
    out_ref[...] = jnp.concatenate(
        [p2n, x1, jnp.zeros((qb, 13), jnp.float32)], axis=1)


def _p2n_call(g2, f1t, x1t, wts):
    n = f1t.shape[0]
    grid = (n // _QB,)
    full = lambda a: pl.BlockSpec(a.shape, lambda i: tuple(0 for _ in a.shape))
    return pl.pallas_call(
        _p2n_body,
        grid=grid,
        in_specs=[
            pl.BlockSpec((_QB * _K, 80), lambda i: (i, 0)),
            pl.BlockSpec((_QB, 64), lambda i: (i, 0)),
            pl.BlockSpec((_QB, 3), lambda i: (i, 0)),
        ] + [full(w) for w in wts],
        out_specs=pl.BlockSpec((_QB, 80), lambda i: (i, 0)),
        out_shape=jax.ShapeDtypeStruct((n, 80), jnp.float32),
    )(g2, f1t, x1t, *wts)


def _n2n_body(g_ref, x1_ref, u1_ref, d1_ref, u2_ref, d2_ref, u3_ref, d3_ref,
              out_ref):
    qb = x1_ref.shape[0]
    g = g_ref[...]
    p2n = g[:, :64]
    xyz = g[:, 64:67]
    xyzn = xyz - _rep_k(x1_ref[...], qb)
    h = _relu(jnp.dot(xyzn, u1_ref[...], preferred_element_type=jnp.float32)
              + d1_ref[...])
    h = _relu(jnp.dot(h, u2_ref[...], preferred_element_type=jnp.float32)
              + d2_ref[...])
    wn = _relu(jnp.dot(h, u3_ref[...], preferred_element_type=jnp.float32)
               + d3_ref[...])
    prod = wn * p2n
    out_ref[...] = jnp.sum(jnp.reshape(prod, (qb, _K, 64)), axis=1)


def _n2n_call(g1, x1t, wts):
    n = x1t.shape[0]
    grid = (n // _QB,)
    full = lambda a: pl.BlockSpec(a.shape, lambda i: tuple(0 for _ in a.shape))
    return pl.pallas_call(
        _n2n_body,
        grid=grid,
        in_specs=[
            pl.BlockSpec((_QB * _K, 80), lambda i: (i, 0)),
            pl.BlockSpec((_QB, 3), lambda i: (i, 0)),
        ] + [full(w) for w in wts],
        out_specs=pl.BlockSpec((_QB, 64), lambda i: (i, 0)),
        out_shape=jax.ShapeDtypeStruct((n, 64), jnp.float32),
    )(g1, x1t, *wts)


def kernel(xyz1, feat1, xyz2, feat2, cost_w1, cost_b1, cost_w2, cost_b2,
           wn1_w1, wn1_b1, wn1_w2, wn1_b2, wn1_w3, wn1_b3,
           wn2_w1, wn2_b1, wn2_w2, wn2_b2, wn2_w3, wn2_b3):
    B, C, N = feat1.shape
    x1t = jnp.transpose(xyz1[0])
    x2t = jnp.transpose(xyz2[0])

    def keys4(xt):
        return jnp.concatenate(
            [2.0 * xt, -jnp.sum(xt * xt, axis=1, keepdims=True)], axis=1)

    n_col = xyz1.shape[2]
    q_aug = jnp.concatenate(
        [xyz1[0], jnp.ones((1, n_col), jnp.float32)], axis=0)
    idx12 = jnp.transpose(_knn2(keys4(x2t), q_aug))
    idx11 = jnp.transpose(_knn2(keys4(x1t), q_aug))

    f2t = jnp.transpose(feat2[0])
    table2 = jnp.concatenate(
        [f2t, x2t, jnp.zeros((N, 13), jnp.float32)], axis=1)
    g2 = _sc_gather(table2, jnp.reshape(idx12, (-1,)))

    f1t = jnp.transpose(feat1[0])
    wts_c = (
        jnp.transpose(cost_w1[:, :64]),
        jnp.transpose(cost_w1[:, 64:128]),
        jnp.transpose(cost_w1[:, 128:131]),
        cost_b1[None, :],
        jnp.transpose(cost_w2), cost_b2[None, :],
        jnp.transpose(wn2_w1), wn2_b1[None, :],
        jnp.transpose(wn2_w2), wn2_b2[None, :],
        jnp.transpose(wn2_w3), wn2_b3[None, :],
    )
    table1 = _p2n_call(g2, f1t, x1t, wts_c)

    g1 = _sc_gather(table1, jnp.reshape(idx11, (-1,)))
    wts_n = (
        jnp.transpose(wn1_w1), wn1_b1[None, :],
        jnp.transpose(wn1_w2), wn1_b2[None, :],
        jnp.transpose(wn1_w3), wn1_b3[None, :],
    )
    n2n = _n2n_call(g1, x1t, wts_n)
    return jnp.transpose(n2n)[None]


# speedup vs baseline: 1.3723x; 1.0420x over previous
import functools

import jax
import jax.numpy as jnp
from jax import lax
from jax.experimental import pallas as pl
from jax.experimental.pallas import tpu as pltpu
from jax.experimental.pallas import tpu_sc as plsc

_K = 16
_QB = 512


_NEG = -3e38
_CH = 128


def _top16(vals, pos, m_sz):
    cols_v, cols_p = [], []
    cur = vals
    for _ in range(_K):
        m = jnp.max(cur, axis=0, keepdims=True)
        cp = jnp.min(jnp.where(cur >= m, pos, m_sz), axis=0, keepdims=True)
        cols_v.append(m)
        cols_p.append(cp)
        cur = jnp.where(pos == cp, _NEG, cur)
    return jnp.concatenate(cols_v, axis=0), jnp.concatenate(cols_p, axis=0)


def _knn2_body(keys_ref, q_ref, idx_ref, s3_ref, cm_ref, rv_ref, ri_ref):
    m_sz = keys_ref.shape[0]
    ql = q_ref.shape[1]
    cc = m_sz // _CH
    keys = keys_ref[...]
    q4 = q_ref[...]
    cross = jnp.dot(keys[:, :3], q4[:3, :],
                    preferred_element_type=jnp.float32)
    s2 = cross + keys[:, 3:4]
    s3_ref[...] = jnp.reshape(s2, (cc, _CH, ql))
    cm_ref[...] = jnp.max(s3_ref[...], axis=1)
    rv_ref[...] = jnp.full((_K, ql), _NEG, jnp.float32)
    ri_ref[...] = jnp.zeros((_K, ql), jnp.int32)
    sub_iota = lax.broadcasted_iota(jnp.int32, (cc, _CH, ql), 1)
    chunk_base = lax.broadcasted_iota(jnp.int32, (cc, ql), 0) * _CH

    def round_body(r, carry):
        cmr = cm_ref[...]
        rv = rv_ref[...]
        rmin = jnp.min(rv, axis=0, keepdims=True)
        amax = jnp.max(cmr - rmin)

        @pl.when(amax >= 0.0)
        def _go():
            s3 = s3_ref[...]
            hit = s3 >= cmr[:, None, :]
            posc = jnp.min(jnp.where(hit, sub_iota, _CH), axis=1)
            snew = jnp.where(sub_iota == posc[:, None, :], _NEG, s3)
            s3_ref[...] = snew
            cm_ref[...] = jnp.max(snew, axis=1)
            pos = chunk_base + posc
            hv, hi = _top16(jnp.concatenate([rv, cmr], axis=0),
                            jnp.concatenate([ri_ref[...], pos], axis=0), m_sz)
            rv_ref[...] = hv
            ri_ref[...] = hi

        return carry

    lax.fori_loop(0, _K, round_body, 0)
    idx_ref[...] = ri_ref[...]


def _knn2(keys4, q_aug):
    m = keys4.shape[0]
    n = q_aug.shape[1]
    cc = m // _CH
    return pl.pallas_call(
        _knn2_body,
        grid=(n // _QB,),
        in_specs=[
            pl.BlockSpec((m, 4), lambda i: (0, 0)),
            pl.BlockSpec((4, _QB), lambda i: (0, i)),
        ],
        out_specs=pl.BlockSpec((_K, _QB), lambda i: (0, i)),
        out_shape=jax.ShapeDtypeStruct((_K, n), jnp.int32),
        scratch_shapes=[
            pltpu.VMEM((cc, _CH, _QB), jnp.float32),
            pltpu.VMEM((cc, _QB), jnp.float32),
            pltpu.VMEM((_K, _QB), jnp.float32),
            pltpu.VMEM((_K, _QB), jnp.int32),
        ],
    )(keys4, q_aug)


def _knn_body(qt_ref, keys_ref, idx_ref):
    q = qt_ref[...]
    keys = keys_ref[...]
    m_sz = keys.shape[1]
    cross = jnp.dot(q, keys, preferred_element_type=jnp.float32)
    k2 = jnp.sum(keys * keys, axis=0, keepdims=True)
    q2 = jnp.sum(q * q, axis=1, keepdims=True)
    s = -((q2 + k2) - 2.0 * cross)
    iota = lax.broadcasted_iota(jnp.int32, (1, m_sz), 1)
    cols = []
    for _ in range(_K):
        m = jnp.max(s, axis=1, keepdims=True)
        hit = s >= m
        pos = jnp.min(jnp.where(hit, iota, m_sz), axis=1, keepdims=True)
        cols.append(pos)
        s = jnp.where(iota == pos, -3e38, s)
    idx_ref[...] = jnp.concatenate(cols, axis=1)


def _knn(query_t, keys):
    n = query_t.shape[0]
    m = keys.shape[1]
    return pl.pallas_call(
        _knn_body,
        grid=(n // _QB,),
        in_specs=[
            pl.BlockSpec((_QB, 3), lambda i: (i, 0)),
            pl.BlockSpec((3, m), lambda i: (0, 0)),
        ],
        out_specs=pl.BlockSpec((_QB, _K), lambda i: (i, 0)),
        out_shape=jax.ShapeDtypeStruct((n, _K), jnp.int32),
    )(query_t, keys)


def _sc_gather(table, idx):
    v, d = table.shape
    bn = idx.shape[0]
    info = plsc.get_sparse_core_info()
    nw = info.num_cores * info.num_subcores
    b_per_w = bn // nw
    ch = min(b_per_w, 512)
    n_ch = b_per_w // ch
    mesh = plsc.VectorSubcoreMesh(core_axis_name="c", subcore_axis_name="s")

    @functools.partial(
        pl.kernel, mesh=mesh,
        out_type=jax.ShapeDtypeStruct((bn, d), jnp.float32),
        compiler_params=pltpu.CompilerParams(use_tc_tiling_on_sc=False),
        scratch_types=[
            pltpu.VMEM((ch,), jnp.int32),
            pltpu.VMEM((ch, d), jnp.float32),
            pltpu.SemaphoreType.DMA,
        ],
    )
    def gk(table_hbm, idx_hbm, out_hbm, idx_v, rows_v, sem):
        wid = lax.axis_index("s") * info.num_cores + lax.axis_index("c")
        base = wid * b_per_w

        def body(i, carry):
            off = base + i * ch
            pltpu.sync_copy(idx_hbm.at[pl.ds(off, ch)], idx_v)
            pltpu.async_copy(table_hbm.at[idx_v], rows_v, sem).wait()
            pltpu.sync_copy(rows_v, out_hbm.at[pl.ds(off, ch)])
            return carry

        lax.fori_loop(0, n_ch, body, 0)

    return gk(table, idx)


def _leaky(x):
    return jnp.where(x >= 0, x, 0.01 * x)


def _relu(x):
    return jnp.maximum(x, 0.0)


def _rep_k(x, qb):
    c = x.shape[1]
    return jnp.reshape(
        jnp.broadcast_to(x[:, None, :], (qb, _K, c)), (qb * _K, c))


def _p2n_body(g_ref, f1_ref, x1_ref,
              w1a_ref, w1b_ref, w1c_ref, b1_ref, w2_ref, b2_ref,
              v1_ref, c1_ref, v2_ref, c2_ref, v3_ref, c3_ref, out_ref):
    qb = f1_ref.shape[0]
    g = g_ref[...]
    f2 = g[:, :64]
    xyz = g[:, 64:67]
    x1 = x1_ref[...]
    xyzn = xyz - _rep_k(x1, qb)
    a1 = jnp.dot(f1_ref[...], w1a_ref[...],
                 preferred_element_type=jnp.float32)
    l1 = _leaky(_rep_k(a1, qb)
                + jnp.dot(f2, w1b_ref[...], preferred_element_type=jnp.float32)
                + jnp.dot(xyzn, w1c_ref[...], preferred_element_type=jnp.float32)
                + b1_ref[...])
    p2p = _leaky(jnp.dot(l1, w2_ref[...], preferred_element_type=jnp.float32)
                 + b2_ref[...])
    h = _relu(jnp.dot(xyzn, v1_ref[...], preferred_element_type=jnp.float32)
              + c1_ref[...])
    h = _relu(jnp.dot(h, v2_ref[...], preferred_element_type=jnp.float32)
              + c2_ref[...])
    wn = _relu(jnp.dot(h, v3_ref[...], preferred_element_type=jnp.float32)
               + c3_ref[...])
    prod = p2p * wn
    p2n = jnp.sum(jnp.reshape(prod, (qb, _K, 64)), axis=1)
    out_ref[...] = jnp.concatenate(
        [p2n, x1, jnp.zeros((qb, 13), jnp.float32)], axis=1)


def _p2n_call(g2, f1t, x1t, wts):
    n = f1t.shape[0]
    grid = (n // _QB,)
    full = lambda a: pl.BlockSpec(a.shape, lambda i: tuple(0 for _ in a.shape))
    return pl.pallas_call(
        _p2n_body,
        grid=grid,
        in_specs=[
            pl.BlockSpec((_QB * _K, 80), lambda i: (i, 0)),
            pl.BlockSpec((_QB, 64), lambda i: (i, 0)),
            pl.BlockSpec((_QB, 3), lambda i: (i, 0)),
        ] + [full(w) for w in wts],
        out_specs=pl.BlockSpec((_QB, 80), lambda i: (i, 0)),
        out_shape=jax.ShapeDtypeStruct((n, 80), jnp.float32),
    )(g2, f1t, x1t, *wts)


def _n2n_body(g_ref, x1_ref, u1_ref, d1_ref, u2_ref, d2_ref, u3_ref, d3_ref,
              out_ref):
    qb = x1_ref.shape[0]
    g = g_ref[...]
    p2n = g[:, :64]
    xyz = g[:, 64:67]
    xyzn = xyz - _rep_k(x1_ref[...], qb)
    h = _relu(jnp.dot(xyzn, u1_ref[...], preferred_element_type=jnp.float32)
              + d1_ref[...])
    h = _relu(jnp.dot(h, u2_ref[...], preferred_element_type=jnp.float32)
              + d2_ref[...])
    wn = _relu(jnp.dot(h, u3_ref[...], preferred_element_type=jnp.float32)
               + d3_ref[...])
    prod = wn * p2n
    out_ref[...] = jnp.sum(jnp.reshape(prod, (qb, _K, 64)), axis=1)


def _n2n_call(g1, x1t, wts):
    n = x1t.shape[0]
    grid = (n // _QB,)
    full = lambda a: pl.BlockSpec(a.shape, lambda i: tuple(0 for _ in a.shape))
    return pl.pallas_call(
        _n2n_body,
        grid=grid,
        in_specs=[
            pl.BlockSpec((_QB * _K, 80), lambda i: (i, 0)),
            pl.BlockSpec((_QB, 3), lambda i: (i, 0)),
        ] + [full(w) for w in wts],
        out_specs=pl.BlockSpec((_QB, 64), lambda i: (i, 0)),
        out_shape=jax.ShapeDtypeStruct((n, 64), jnp.float32),
    )(g1, x1t, *wts)


def kernel(xyz1, feat1, xyz2, feat2, cost_w1, cost_b1, cost_w2, cost_b2,
           wn1_w1, wn1_b1, wn1_w2, wn1_b2, wn1_w3, wn1_b3,
           wn2_w1, wn2_b1, wn2_w2, wn2_b2, wn2_w3, wn2_b3):
    B, C, N = feat1.shape
    x1t = jnp.transpose(xyz1[0])
    x2t = jnp.transpose(xyz2[0])

    def keys4(xt):
        return jnp.concatenate(
            [2.0 * xt, -jnp.sum(xt * xt, axis=1, keepdims=True)], axis=1)

    n_col = xyz1.shape[2]
    q_aug = jnp.concatenate(
        [xyz1[0], jnp.ones((1, n_col), jnp.float32)], axis=0)
    idx12 = jnp.transpose(_knn2(keys4(x2t), q_aug))
    idx11 = jnp.transpose(_knn2(keys4(x1t), q_aug))

    f2t = jnp.transpose(feat2[0])
    table2 = jnp.concatenate(
        [f2t, x2t, jnp.zeros((N, 13), jnp.float32)], axis=1)
    g2 = _sc_gather(table2, jnp.reshape(idx12, (-1,)))

    f1t = jnp.transpose(feat1[0])
    wts_c = (
        jnp.transpose(cost_w1[:, :64]),
        jnp.transpose(cost_w1[:, 64:128]),
        jnp.transpose(cost_w1[:, 128:131]),
        cost_b1[None, :],
        jnp.transpose(cost_w2), cost_b2[None, :],
        jnp.transpose(wn2_w1), wn2_b1[None, :],
        jnp.transpose(wn2_w2), wn2_b2[None, :],
        jnp.transpose(wn2_w3), wn2_b3[None, :],
    )
    table1 = _p2n_call(g2, f1t, x1t, wts_c)

    g1 = _sc_gather(table1, jnp.reshape(idx11, (-1,)))
    wts_n = (
        jnp.transpose(wn1_w1), wn1_b1[None, :],
        jnp.transpose(wn1_w2), wn1_b2[None, :],
        jnp.transpose(wn1_w3), wn1_b3[None, :],
    )
    n2n = _n2n_call(g1, x1t, wts_n)
    return jnp.transpose(n2n)[None]
